# Initial kernel scaffold; baseline (speedup 1.0000x reference)
import jax
import jax.numpy as jnp
from jax.experimental import pallas as pl


def kernel(x, edge_index, edge_attr, batch, nonring, W0, b0, We1, be1, We2, be2, root, bconv, Wih, bih, Whh, bhh, Wlih, blih, Wlhh, blhh, W1, b1, W2, b2):
    raise NotImplementedError("write your pallas kernel here")


# speedup vs baseline: 1.2606x; 1.2606x over previous
import functools

import jax
import jax.numpy as jnp
from jax import lax
from jax.experimental import pallas as pl
from jax.experimental.pallas import tpu as pltpu
from jax.experimental.pallas import tpu_sc as plsc

N = 10000
E = 160000
D = 16
NW = 32
E_PER_W = 5120
E_PAD = NW * E_PER_W
CHUNK = 128
N_CHUNKS = E_PER_W // CHUNK
N_PAD = 10016
ROWS_PER_TILE = N_PAD // 16


def _mesh():
    return plsc.VectorSubcoreMesh(core_axis_name="c", subcore_axis_name="s")


def _make_gather(per_w, chunk):
    n_chunks = per_w // chunk

    @functools.partial(
        pl.kernel,
        mesh=_mesh(),
        compiler_params=pltpu.CompilerParams(use_tc_tiling_on_sc=False),
        out_type=jax.ShapeDtypeStruct((NW * per_w, D), jnp.float32),
        scratch_types=[
            pltpu.VMEM((per_w,), jnp.int32),
            pltpu.VMEM((per_w, D), jnp.float32),
            pltpu.SemaphoreType.DMA,
        ],
    )
    def gk(table_hbm, idx_hbm, out_hbm, idx_v, rows_v, sem):
        wid = lax.axis_index("s") * 2 + lax.axis_index("c")
        base = wid * per_w
        pltpu.sync_copy(idx_hbm.at[pl.ds(base, per_w)], idx_v)

        def body(j, c):
            off = pl.multiple_of(j * chunk, 8)
            pltpu.async_copy(
                table_hbm.at[idx_v.at[pl.ds(off, chunk)]],
                rows_v.at[pl.ds(off, chunk)],
                sem,
            ).wait()
            return c

        lax.fori_loop(0, n_chunks, body, 0)
        pltpu.sync_copy(rows_v, out_hbm.at[pl.ds(base, per_w)])

    return gk


@functools.partial(
    pl.kernel,
    mesh=_mesh(),
    compiler_params=pltpu.CompilerParams(use_tc_tiling_on_sc=False),
    out_type=jax.ShapeDtypeStruct((2, N_PAD, D), jnp.float32),
    scratch_types=[
        pltpu.VMEM((N_CHUNKS, CHUNK), jnp.int32),
        pltpu.VMEM((E_PER_W, D), jnp.float32),
        pltpu.VMEM_SHARED((N_PAD, D), jnp.float32),
    ],
)
def _scatter_add(vals_hbm, idx_hbm, zeros_hbm, out_hbm, idx_v, vals_v, shared):
    c = lax.axis_index("c")
    s = lax.axis_index("s")
    wid = s * 2 + c
    base = wid * E_PER_W
    r0 = s * ROWS_PER_TILE
    pltpu.sync_copy(
        zeros_hbm.at[pl.ds(r0, ROWS_PER_TILE)], shared.at[pl.ds(r0, ROWS_PER_TILE)]
    )
    pltpu.sync_copy(vals_hbm.at[pl.ds(base, E_PER_W)], vals_v)
    pltpu.sync_copy(idx_hbm.at[wid], idx_v)
    plsc.subcore_barrier()

    def body(j, carry):
        off = pl.multiple_of(j * CHUNK, 8)
        pltpu.sync_copy(
            vals_v.at[pl.ds(off, CHUNK)], shared.at[idx_v.at[j]], add=True
        )
        return carry

    lax.fori_loop(0, N_CHUNKS, body, 0)
    plsc.subcore_barrier()
    pltpu.sync_copy(
        shared.at[pl.ds(r0, ROWS_PER_TILE)], out_hbm.at[c, pl.ds(r0, ROWS_PER_TILE)]
    )


_EBLK = 2048


def _f_body(ea_ref, w_ref, b_ref, o_ref):
    o_ref[...] = jnp.maximum(
        jnp.dot(ea_ref[...], w_ref[...], preferred_element_type=jnp.float32)
        + b_ref[...],
        0.0,
    )


def _msg_body(u_ref, f_ref, g2_ref, b2_ref, o_ref):
    u = u_ref[...]
    f = f_ref[...]
    p = jnp.concatenate([f[:, k : k + 1] * u for k in range(D)], axis=1)
    o_ref[...] = jnp.dot(
        p, g2_ref[...], preferred_element_type=jnp.float32
    ) + jnp.dot(u, b2_ref[...], preferred_element_type=jnp.float32)


def _s0_body(x_ref, w_ref, b_ref, o_ref):
    o_ref[...] = jnp.maximum(
        jnp.dot(x_ref[...], w_ref[...], preferred_element_type=jnp.float32)
        + b_ref[...],
        0.0,
    )


def _deg_body(d0_ref, d1_ref, o_ref):
    o_ref[...] = jnp.maximum(d0_ref[...] + d1_ref[...], 1.0)


def _gru_body(s_ref, a0_ref, a1_ref, deg_ref, root_ref, bconv_ref, wih_ref,
              bih_ref, whh_ref, bhh_ref, o_ref):
    s = s_ref[...]
    aggr = (a0_ref[...] + a1_ref[...]) / deg_ref[...]
    m = jnp.maximum(
        jnp.dot(s, root_ref[...], preferred_element_type=jnp.float32)
        + aggr + bconv_ref[...],
        0.0,
    )
    gi = jnp.dot(m, wih_ref[...], preferred_element_type=jnp.float32) + bih_ref[...]
    gh = jnp.dot(s, whh_ref[...], preferred_element_type=jnp.float32) + bhh_ref[...]
    rg = jax.nn.sigmoid(gi[:, 0:D] + gh[:, 0:D])
    zg = jax.nn.sigmoid(gi[:, D : 2 * D] + gh[:, D : 2 * D])
    ng = jnp.tanh(gi[:, 2 * D : 3 * D] + rg * gh[:, 2 * D : 3 * D])
    o_ref[...] = (1.0 - zg) * ng + zg * s


def _set2set_body(s_ref, wlih_ref, blih_ref, wlhh_ref, blhh_ref, o_ref):
    s = s_ref[...]
    wlih = wlih_ref[...]
    blih = blih_ref[...]
    wlhh = wlhh_ref[...]
    blhh = blhh_ref[...]
    hl = jnp.zeros((1, D), jnp.float32)
    cl = jnp.zeros((1, D), jnp.float32)
    q_star = jnp.zeros((1, 2 * D), jnp.float32)
    for _ in range(6):
        gates = (
            jnp.dot(q_star, wlih, preferred_element_type=jnp.float32) + blih
            + jnp.dot(hl, wlhh, preferred_element_type=jnp.float32) + blhh
        )
        ig = jax.nn.sigmoid(gates[:, 0:D])
        fg = jax.nn.sigmoid(gates[:, D : 2 * D])
        gg = jnp.tanh(gates[:, 2 * D : 3 * D])
        og = jax.nn.sigmoid(gates[:, 3 * D : 4 * D])
        cl = fg * cl + ig * gg
        hl = og * jnp.tanh(cl)
        e = jnp.sum(s * hl, axis=1, keepdims=True)
        emax = jnp.max(e)
        a = jnp.exp(e - emax)
        a = a / jnp.sum(a)
        r = jnp.sum(a * s, axis=0, keepdims=True)
        q_star = jnp.concatenate([hl, r], axis=1)
    o_ref[...] = q_star


def _final_body(z_ref, w1_ref, b1_ref, w2_ref, b2_ref, o_ref):
    h = jnp.maximum(
        jnp.dot(z_ref[...], w1_ref[...], preferred_element_type=jnp.float32)
        + b1_ref[...],
        0.0,
    )
    o_ref[...] = (
        jnp.dot(h, w2_ref[...], preferred_element_type=jnp.float32) + b2_ref[...]
    )


def _full(shape):
    return pl.BlockSpec(shape, lambda *_: tuple(0 for _ in shape))


def kernel(x, edge_index, edge_attr, batch, nonring, W0, b0, We1, be1, We2,
           be2, root, bconv, Wih, bih, Whh, bhh, Wlih, blih, Wlhh, blhh, W1,
           b1, W2, b2):
    f32 = jnp.float32
    src = edge_index[0].astype(jnp.int32)
    dst = edge_index[1].astype(jnp.int32)
    pad = E_PAD - E
    src_p = jnp.concatenate([src, jnp.zeros((pad,), jnp.int32)])
    dst_p = jnp.concatenate(
        [dst, jnp.full((pad,), N, jnp.int32)]
    ).reshape(NW, N_CHUNKS, CHUNK)
    ea_p = jnp.concatenate([edge_attr.astype(f32), jnp.zeros((pad, 7), f32)])
    zeros_np = jnp.zeros((N_PAD, D), f32)
    ones_e = jnp.ones((E_PAD, D), f32)
    G2 = We2.reshape(D * D, D)
    B2 = be2.reshape(D, D)

    gather_edges = _make_gather(E_PER_W, CHUNK)
    gather_sel = _make_gather(2048 // NW, 2048 // NW)

    nblk = E_PAD // _EBLK
    f = pl.pallas_call(
        _f_body,
        grid=(nblk,),
        in_specs=[
            pl.BlockSpec((_EBLK, 7), lambda i: (i, 0)),
            _full((7, D)),
            _full((1, D)),
        ],
        out_specs=pl.BlockSpec((_EBLK, D), lambda i: (i, 0)),
        out_shape=jax.ShapeDtypeStruct((E_PAD, D), f32),
    )(ea_p, We1, be1.reshape(1, D))

    s = pl.pallas_call(
        _s0_body,
        in_specs=[_full((N, 3)), _full((3, D)), _full((1, D))],
        out_specs=_full((N, D)),
        out_shape=jax.ShapeDtypeStruct((N, D), f32),
    )(x.astype(f32), W0, b0.reshape(1, D))

    degp = _scatter_add(ones_e, dst_p, zeros_np)
    deg16 = pl.pallas_call(
        _deg_body,
        in_specs=[_full((N, D)), _full((N, D))],
        out_specs=_full((N, D)),
        out_shape=jax.ShapeDtypeStruct((N, D), f32),
    )(degp[0, :N], degp[1, :N])

    msg_call = pl.pallas_call(
        _msg_body,
        grid=(nblk,),
        in_specs=[
            pl.BlockSpec((_EBLK, D), lambda i: (i, 0)),
            pl.BlockSpec((_EBLK, D), lambda i: (i, 0)),
            _full((D * D, D)),
            _full((D, D)),
        ],
        out_specs=pl.BlockSpec((_EBLK, D), lambda i: (i, 0)),
        out_shape=jax.ShapeDtypeStruct((E_PAD, D), f32),
    )

    gru_call = pl.pallas_call(
        _gru_body,
        in_specs=[
            _full((N, D)), _full((N, D)), _full((N, D)), _full((N, D)),
            _full((D, D)), _full((1, D)), _full((D, 3 * D)), _full((1, 3 * D)),
            _full((D, 3 * D)), _full((1, 3 * D)),
        ],
        out_specs=_full((N, D)),
        out_shape=jax.ShapeDtypeStruct((N, D), f32),
    )

    for _ in range(6):
        u = gather_edges(s, src_p)
        msg = msg_call(u, f, G2, B2)
        ap = _scatter_add(msg, dst_p, zeros_np)
        s = gru_call(s, ap[0, :N], ap[1, :N], deg16, root,
                     bconv.reshape(1, D), Wih, bih.reshape(1, 3 * D), Whh,
                     bhh.reshape(1, 3 * D))

    pool = pl.pallas_call(
        _set2set_body,
        in_specs=[
            _full((N, D)), _full((2 * D, 4 * D)), _full((1, 4 * D)),
            _full((D, 4 * D)), _full((1, 4 * D)),
        ],
        out_specs=_full((1, 2 * D)),
        out_shape=jax.ShapeDtypeStruct((1, 2 * D), f32),
    )(s, Wlih, blih.reshape(1, 4 * D), Wlhh, blhh.reshape(1, 4 * D))

    sel = gather_sel(s, nonring.reshape(-1).astype(jnp.int32))
    sel_t = sel.reshape(4 * D, -1).T
    kk = sel_t.shape[0]
    rep = jnp.repeat(pool.reshape(-1), kk).reshape(kk, -1)
    z = jnp.concatenate([sel_t, rep], axis=1)

    return pl.pallas_call(
        _final_body,
        in_specs=[
            _full((kk, 6 * D)), _full((6 * D, D)), _full((1, D)),
            _full((D, 6)), _full((1, 6)),
        ],
        out_specs=_full((kk, 6)),
        out_shape=jax.ShapeDtypeStruct((kk, 6), f32),
    )(z, W1, b1.reshape(1, D), W2, b2.reshape(1, 6))


# speedup vs baseline: 1.2761x; 1.0124x over previous
import functools

import jax
import jax.numpy as jnp
from jax import lax
from jax.experimental import pallas as pl
from jax.experimental.pallas import tpu as pltpu
from jax.experimental.pallas import tpu_sc as plsc

N = 10000
E = 160000
D = 16
NW = 32
E_PER_W = 5120
E_PAD = NW * E_PER_W
CHUNK = 128
N_CHUNKS = E_PER_W // CHUNK
N_PAD = 10016
ROWS_PER_TILE = N_PAD // 16


def _mesh():
    return plsc.VectorSubcoreMesh(core_axis_name="c", subcore_axis_name="s")


def _make_gather(per_w, chunk):
    n_chunks = per_w // chunk

    @functools.partial(
        pl.kernel,
        mesh=_mesh(),
        compiler_params=pltpu.CompilerParams(use_tc_tiling_on_sc=False),
        out_type=jax.ShapeDtypeStruct((NW * per_w, D), jnp.float32),
        scratch_types=[
            pltpu.VMEM((per_w,), jnp.int32),
            pltpu.VMEM((per_w, D), jnp.float32),
            pltpu.SemaphoreType.DMA,
        ],
    )
    def gk(table_hbm, idx_hbm, out_hbm, idx_v, rows_v, sem):
        wid = lax.axis_index("s") * 2 + lax.axis_index("c")
        base = wid * per_w
        pltpu.sync_copy(idx_hbm.at[pl.ds(base, per_w)], idx_v)

        for j in range(n_chunks):
            pltpu.async_copy(
                table_hbm.at[idx_v.at[pl.ds(j * chunk, chunk)]],
                rows_v.at[pl.ds(j * chunk, chunk)],
                sem,
            )
        pltpu.make_async_copy(
            out_hbm.at[pl.ds(base, per_w)], rows_v, sem
        ).wait()
        pltpu.sync_copy(rows_v, out_hbm.at[pl.ds(base, per_w)])

    return gk


@functools.partial(
    pl.kernel,
    mesh=_mesh(),
    compiler_params=pltpu.CompilerParams(use_tc_tiling_on_sc=False),
    out_type=jax.ShapeDtypeStruct((2, N_PAD, D), jnp.float32),
    scratch_types=[
        pltpu.VMEM((N_CHUNKS, CHUNK), jnp.int32),
        pltpu.VMEM((E_PER_W, D), jnp.float32),
        pltpu.VMEM_SHARED((N_PAD, D), jnp.float32),
        pltpu.SemaphoreType.DMA,
    ],
)
def _scatter_add(vals_hbm, idx_hbm, zeros_hbm, out_hbm, idx_v, vals_v, shared,
                 sem):
    c = lax.axis_index("c")
    s = lax.axis_index("s")
    wid = s * 2 + c
    base = wid * E_PER_W
    r0 = s * ROWS_PER_TILE
    pltpu.sync_copy(
        zeros_hbm.at[pl.ds(r0, ROWS_PER_TILE)], shared.at[pl.ds(r0, ROWS_PER_TILE)]
    )
    pltpu.sync_copy(vals_hbm.at[pl.ds(base, E_PER_W)], vals_v)
    pltpu.sync_copy(idx_hbm.at[wid], idx_v)
    plsc.subcore_barrier()

    for j in range(N_CHUNKS):
        pltpu.async_copy(
            vals_v.at[pl.ds(j * CHUNK, CHUNK)], shared.at[idx_v.at[j]], sem,
            add=True,
        )
    pltpu.make_async_copy(vals_hbm.at[pl.ds(base, E_PER_W)], vals_v, sem).wait()
    plsc.subcore_barrier()
    pltpu.sync_copy(
        shared.at[pl.ds(r0, ROWS_PER_TILE)], out_hbm.at[c, pl.ds(r0, ROWS_PER_TILE)]
    )


_EBLK = 2048


def _f_body(ea_ref, w_ref, b_ref, o_ref):
    o_ref[...] = jnp.maximum(
        jnp.dot(ea_ref[...], w_ref[...], preferred_element_type=jnp.float32)
        + b_ref[...],
        0.0,
    )


def _msg_body(u_ref, f_ref, g2_ref, b2_ref, o_ref):
    u = u_ref[...]
    f = f_ref[...]
    p = jnp.concatenate([f[:, k : k + 1] * u for k in range(D)], axis=1)
    o_ref[...] = jnp.dot(
        p, g2_ref[...], preferred_element_type=jnp.float32
    ) + jnp.dot(u, b2_ref[...], preferred_element_type=jnp.float32)


def _s0_body(x_ref, w_ref, b_ref, o_ref):
    o_ref[...] = jnp.maximum(
        jnp.dot(x_ref[...], w_ref[...], preferred_element_type=jnp.float32)
        + b_ref[...],
        0.0,
    )


def _deg_body(d0_ref, d1_ref, o_ref):
    o_ref[...] = jnp.maximum(d0_ref[...] + d1_ref[...], 1.0)


def _gru_body(s_ref, a0_ref, a1_ref, deg_ref, root_ref, bconv_ref, wih_ref,
              bih_ref, whh_ref, bhh_ref, o_ref):
    s = s_ref[...]
    aggr = (a0_ref[...] + a1_ref[...]) / deg_ref[...]
    m = jnp.maximum(
        jnp.dot(s, root_ref[...], preferred_element_type=jnp.float32)
        + aggr + bconv_ref[...],
        0.0,
    )
    gi = jnp.dot(m, wih_ref[...], preferred_element_type=jnp.float32) + bih_ref[...]
    gh = jnp.dot(s, whh_ref[...], preferred_element_type=jnp.float32) + bhh_ref[...]
    rg = jax.nn.sigmoid(gi[:, 0:D] + gh[:, 0:D])
    zg = jax.nn.sigmoid(gi[:, D : 2 * D] + gh[:, D : 2 * D])
    ng = jnp.tanh(gi[:, 2 * D : 3 * D] + rg * gh[:, 2 * D : 3 * D])
    o_ref[...] = (1.0 - zg) * ng + zg * s


def _set2set_body(s_ref, wlih_ref, blih_ref, wlhh_ref, blhh_ref, o_ref):
    s = s_ref[...]
    wlih = wlih_ref[...]
    blih = blih_ref[...]
    wlhh = wlhh_ref[...]
    blhh = blhh_ref[...]
    hl = jnp.zeros((1, D), jnp.float32)
    cl = jnp.zeros((1, D), jnp.float32)
    q_star = jnp.zeros((1, 2 * D), jnp.float32)
    for _ in range(6):
        gates = (
            jnp.dot(q_star, wlih, preferred_element_type=jnp.float32) + blih
            + jnp.dot(hl, wlhh, preferred_element_type=jnp.float32) + blhh
        )
        ig = jax.nn.sigmoid(gates[:, 0:D])
        fg = jax.nn.sigmoid(gates[:, D : 2 * D])
        gg = jnp.tanh(gates[:, 2 * D : 3 * D])
        og = jax.nn.sigmoid(gates[:, 3 * D : 4 * D])
        cl = fg * cl + ig * gg
        hl = og * jnp.tanh(cl)
        e = jnp.sum(s * hl, axis=1, keepdims=True)
        emax = jnp.max(e)
        a = jnp.exp(e - emax)
        a = a / jnp.sum(a)
        r = jnp.sum(a * s, axis=0, keepdims=True)
        q_star = jnp.concatenate([hl, r], axis=1)
    o_ref[...] = q_star


def _final_body(z_ref, w1_ref, b1_ref, w2_ref, b2_ref, o_ref):
    h = jnp.maximum(
        jnp.dot(z_ref[...], w1_ref[...], preferred_element_type=jnp.float32)
        + b1_ref[...],
        0.0,
    )
    o_ref[...] = (
        jnp.dot(h, w2_ref[...], preferred_element_type=jnp.float32) + b2_ref[...]
    )


def _full(shape):
    return pl.BlockSpec(shape, lambda *_: tuple(0 for _ in shape))


def kernel(x, edge_index, edge_attr, batch, nonring, W0, b0, We1, be1, We2,
           be2, root, bconv, Wih, bih, Whh, bhh, Wlih, blih, Wlhh, blhh, W1,
           b1, W2, b2):
    f32 = jnp.float32
    src = edge_index[0].astype(jnp.int32)
    dst = edge_index[1].astype(jnp.int32)
    pad = E_PAD - E
    src_p = jnp.concatenate([src, jnp.zeros((pad,), jnp.int32)])
    dst_p = jnp.concatenate(
        [dst, jnp.full((pad,), N, jnp.int32)]
    ).reshape(NW, N_CHUNKS, CHUNK)
    ea_p = jnp.concatenate([edge_attr.astype(f32), jnp.zeros((pad, 7), f32)])
    zeros_np = jnp.zeros((N_PAD, D), f32)
    ones_e = jnp.ones((E_PAD, D), f32)
    G2 = We2.reshape(D * D, D)
    B2 = be2.reshape(D, D)

    gather_edges = _make_gather(E_PER_W, CHUNK)
    gather_sel = _make_gather(2048 // NW, 2048 // NW)

    nblk = E_PAD // _EBLK
    f = pl.pallas_call(
        _f_body,
        grid=(nblk,),
        in_specs=[
            pl.BlockSpec((_EBLK, 7), lambda i: (i, 0)),
            _full((7, D)),
            _full((1, D)),
        ],
        out_specs=pl.BlockSpec((_EBLK, D), lambda i: (i, 0)),
        out_shape=jax.ShapeDtypeStruct((E_PAD, D), f32),
    )(ea_p, We1, be1.reshape(1, D))

    s = pl.pallas_call(
        _s0_body,
        in_specs=[_full((N, 3)), _full((3, D)), _full((1, D))],
        out_specs=_full((N, D)),
        out_shape=jax.ShapeDtypeStruct((N, D), f32),
    )(x.astype(f32), W0, b0.reshape(1, D))

    degp = _scatter_add(ones_e, dst_p, zeros_np)
    deg16 = pl.pallas_call(
        _deg_body,
        in_specs=[_full((N, D)), _full((N, D))],
        out_specs=_full((N, D)),
        out_shape=jax.ShapeDtypeStruct((N, D), f32),
    )(degp[0, :N], degp[1, :N])

    msg_call = pl.pallas_call(
        _msg_body,
        grid=(nblk,),
        in_specs=[
            pl.BlockSpec((_EBLK, D), lambda i: (i, 0)),
            pl.BlockSpec((_EBLK, D), lambda i: (i, 0)),
            _full((D * D, D)),
            _full((D, D)),
        ],
        out_specs=pl.BlockSpec((_EBLK, D), lambda i: (i, 0)),
        out_shape=jax.ShapeDtypeStruct((E_PAD, D), f32),
    )

    gru_call = pl.pallas_call(
        _gru_body,
        in_specs=[
            _full((N, D)), _full((N, D)), _full((N, D)), _full((N, D)),
            _full((D, D)), _full((1, D)), _full((D, 3 * D)), _full((1, 3 * D)),
            _full((D, 3 * D)), _full((1, 3 * D)),
        ],
        out_specs=_full((N, D)),
        out_shape=jax.ShapeDtypeStruct((N, D), f32),
    )

    for _ in range(6):
        u = gather_edges(s, src_p)
        msg = msg_call(u, f, G2, B2)
        ap = _scatter_add(msg, dst_p, zeros_np)
        s = gru_call(s, ap[0, :N], ap[1, :N], deg16, root,
                     bconv.reshape(1, D), Wih, bih.reshape(1, 3 * D), Whh,
                     bhh.reshape(1, 3 * D))

    pool = pl.pallas_call(
        _set2set_body,
        in_specs=[
            _full((N, D)), _full((2 * D, 4 * D)), _full((1, 4 * D)),
            _full((D, 4 * D)), _full((1, 4 * D)),
        ],
        out_specs=_full((1, 2 * D)),
        out_shape=jax.ShapeDtypeStruct((1, 2 * D), f32),
    )(s, Wlih, blih.reshape(1, 4 * D), Wlhh, blhh.reshape(1, 4 * D))

    sel = gather_sel(s, nonring.reshape(-1).astype(jnp.int32))
    sel_t = sel.reshape(4 * D, -1).T
    kk = sel_t.shape[0]
    rep = jnp.repeat(pool.reshape(-1), kk).reshape(kk, -1)
    z = jnp.concatenate([sel_t, rep], axis=1)

    return pl.pallas_call(
        _final_body,
        in_specs=[
            _full((kk, 6 * D)), _full((6 * D, D)), _full((1, D)),
            _full((D, 6)), _full((1, 6)),
        ],
        out_specs=_full((kk, 6)),
        out_shape=jax.ShapeDtypeStruct((kk, 6), f32),
    )(z, W1, b1.reshape(1, D), W2, b2.reshape(1, 6))
